# baseline (speedup 1.0000x reference)
import functools

import jax
import jax.numpy as jnp
from jax import lax
from jax.experimental import pallas as pl
from jax.experimental.pallas import tpu as pltpu
from jax.experimental.pallas import tpu_sc as plsc

N_NODES = 10000
D = 256
XPAD = 10240
E_TOT = N_NODES + 160000

CA = 96
NCH_A = 56
EPW_A = CA * NCH_A
E_PAD = 32 * EPW_A

CB = 64
NCH_B = 168
EPW_B = CB * NCH_B

PAD_IDX = 10008

_MESH = plsc.VectorSubcoreMesh(core_axis_name="c", subcore_axis_name="s")
_SC_PARAMS = pltpu.CompilerParams(needs_layout_passes=False)


def _proj_body(x_ref, wl_ref, bl_ref, wr_ref, br_ref, xl_ref, xr_ref):
    xb = x_ref[...]
    xl_ref[...] = jnp.dot(xb, wl_ref[...], preferred_element_type=jnp.float32) + bl_ref[...]
    xr_ref[...] = jnp.dot(xb, wr_ref[...], preferred_element_type=jnp.float32) + br_ref[...]


def _project(x_pad, W_l, b_l, W_r, b_r):
    blk = 1024
    return pl.pallas_call(
        _proj_body,
        grid=(XPAD // blk,),
        in_specs=[
            pl.BlockSpec((blk, D), lambda i: (i, 0)),
            pl.BlockSpec((D, D), lambda i: (0, 0)),
            pl.BlockSpec((D,), lambda i: (0,)),
            pl.BlockSpec((D, D), lambda i: (0, 0)),
            pl.BlockSpec((D,), lambda i: (0,)),
        ],
        out_specs=[
            pl.BlockSpec((blk, D), lambda i: (i, 0)),
            pl.BlockSpec((blk, D), lambda i: (i, 0)),
        ],
        out_shape=[
            jax.ShapeDtypeStruct((XPAD, D), jnp.float32),
            jax.ShapeDtypeStruct((XPAD, D), jnp.float32),
        ],
    )(x_pad, W_l, b_l, W_r, b_r)


def _logits_body(xl, xr, src, dst, att, w_out, s2_out,
                 idxs_v, idxd_v, xlr, xrr, w2, dstc, att_v, zbuf, tbuf, s_sh,
                 semgl, semgr, sems, semw):
    cid = lax.axis_index("c")
    sid = lax.axis_index("s")
    wid = cid * 16 + sid
    ebase = wid * EPW_A

    for i in range(40):
        zbuf[pl.ds(i * 16, 16)] = jnp.zeros((16,), jnp.float32)
    pltpu.sync_copy(zbuf, s_sh.at[pl.ds(sid * 640, 640)])
    pltpu.sync_copy(att, att_v)
    pltpu.sync_copy(src.at[pl.ds(ebase, EPW_A)], idxs_v)
    pltpu.sync_copy(dst.at[pl.ds(ebase, EPW_A)], idxd_v)
    plsc.subcore_barrier()

    def gather_pair(c, b):
        pltpu.async_copy(xl.at[idxs_v.at[pl.ds(c * CA, CA)]], xlr.at[b], semgl.at[b])
        pltpu.async_copy(xr.at[idxd_v.at[pl.ds(c * CA, CA)]], xrr.at[b], semgr.at[b])

    for b in range(2):
        gather_pair(b, b)

    lanes16 = lax.iota(jnp.int32, 16) * 16

    def pair(g, _):
        for b in range(2):
            c = 2 * g + b
            pltpu.make_async_copy(
                xl.at[idxs_v.at[pl.ds(c * CA, CA)]], xlr.at[b], semgl.at[b]).wait()
            pltpu.make_async_copy(
                xr.at[idxd_v.at[pl.ds(c * CA, CA)]], xrr.at[b], semgr.at[b]).wait()

            @pl.when(g >= 1)
            def _():
                cp = c - 2
                pltpu.make_async_copy(
                    w2.at[b], s_sh.at[dstc.at[b]], sems.at[b]).wait()
                pltpu.make_async_copy(
                    w2.at[b], w_out.at[pl.ds(ebase + cp * CA, CA)], semw.at[b]).wait()

            xb = xlr.at[b]
            rb = xrr.at[b]

            def edge_group(q, _):
                for j in range(16):
                    e = q * 16 + j
                    acc = jnp.zeros((16,), jnp.float32)
                    for k in range(16):
                        z = xb[e, pl.ds(k * 16, 16)] + rb[e, pl.ds(k * 16, 16)]
                        acc = acc + jnp.maximum(z, 0.2 * z) * att_v[pl.ds(k * 16, 16)]
                    plsc.store_scatter(tbuf, [lanes16 + j], acc)
                vec = tbuf[pl.ds(0, 16)]
                for k in range(1, 16):
                    vec = vec + tbuf[pl.ds(k * 16, 16)]
                w2[b, pl.ds(q * 16, 16)] = jnp.exp(vec)
                return 0

            lax.fori_loop(0, CA // 16, edge_group, 0)

            for i in range(CA // 16):
                sl = pl.ds(i * 16, 16)
                dstc[b, sl] = idxd_v[pl.ds(c * CA + i * 16, 16)]
            pltpu.async_copy(
                w2.at[b], s_sh.at[dstc.at[b]], sems.at[b], add=True)
            pltpu.async_copy(
                w2.at[b], w_out.at[pl.ds(ebase + c * CA, CA)], semw.at[b])

            @pl.when(c + 2 < NCH_A)
            def _():
                gather_pair(c + 2, b)

        return 0

    lax.fori_loop(0, NCH_A // 2, pair, 0)

    for b in range(2):
        c = NCH_A - 2 + b
        pltpu.make_async_copy(
            w2.at[b], s_sh.at[dstc.at[b]], sems.at[b]).wait()
        pltpu.make_async_copy(
            w2.at[b], w_out.at[pl.ds(ebase + c * CA, CA)], semw.at[b]).wait()
    plsc.subcore_barrier()

    @pl.when(sid == 0)
    def _():
        pltpu.sync_copy(s_sh, s2_out.at[pl.ds(cid * XPAD, XPAD)])


def _run_logits(xl, xr, src, dst, att):
    return pl.kernel(
        _logits_body,
        out_type=[
            jax.ShapeDtypeStruct((E_PAD,), jnp.float32),
            jax.ShapeDtypeStruct((2 * XPAD,), jnp.float32),
        ],
        mesh=_MESH,
        scratch_types=[
            pltpu.VMEM((EPW_A,), jnp.int32),
            pltpu.VMEM((EPW_A,), jnp.int32),
            pltpu.VMEM((2, CA, D), jnp.float32),
            pltpu.VMEM((2, CA, D), jnp.float32),
            pltpu.VMEM((2, CA), jnp.float32),
            pltpu.VMEM((2, CA), jnp.int32),
            pltpu.VMEM((D,), jnp.float32),
            pltpu.VMEM((640,), jnp.float32),
            pltpu.VMEM((D,), jnp.float32),
            pltpu.VMEM_SHARED((XPAD,), jnp.float32),
            pltpu.SemaphoreType.DMA((2,)),
            pltpu.SemaphoreType.DMA((2,)),
            pltpu.SemaphoreType.DMA((2,)),
            pltpu.SemaphoreType.DMA((2,)),
        ],
        compiler_params=_SC_PARAMS,
    )(xl, xr, src, dst, att)


def _agg_body(xlh, src, dst, w, s2, bias, out,
              rows, sbuf, scb, dcb, wcb, dstc, s0_v, iv_v, bias_v,
              acc_sh, si, sg, ss):
    cid = lax.axis_index("c")
    sid = lax.axis_index("s")
    roff = cid * XPAD
    ebase = sid * EPW_B
    NG = CB // 16
    zero16 = jnp.zeros((16,), jnp.int32)

    def zrow(j, _):
        for k in range(8):
            rows[0, j, pl.ds(k * 16, 16)] = jnp.zeros((16,), jnp.float32)
        return 0

    lax.fori_loop(0, CB, zrow, 0)
    for j in range(10):
        pltpu.sync_copy(rows.at[0], acc_sh.at[pl.ds(sid * 640 + j * 64, 64)])
    pltpu.sync_copy(bias.at[pl.ds(cid * 128, 128)], bias_v)
    plsc.subcore_barrier()

    def ring_issue(c, s, sync=False):
        sl = pl.ds(s * CB, CB)
        esl = pl.ds(ebase + c * CB, CB)
        if sync:
            pltpu.sync_copy(src.at[esl], scb.at[sl])
            pltpu.sync_copy(dst.at[esl], dcb.at[sl])
            pltpu.sync_copy(w.at[esl], wcb.at[sl])
        else:
            pltpu.async_copy(src.at[esl], scb.at[sl], si.at[s])
            pltpu.async_copy(dst.at[esl], dcb.at[sl], si.at[s])
            pltpu.async_copy(w.at[esl], wcb.at[sl], si.at[s])

    def ring_wait(c, s):
        sl = pl.ds(s * CB, CB)
        esl = pl.ds(ebase + c * CB, CB)
        pltpu.make_async_copy(src.at[esl], scb.at[sl], si.at[s]).wait()
        pltpu.make_async_copy(dst.at[esl], dcb.at[sl], si.at[s]).wait()
        pltpu.make_async_copy(w.at[esl], wcb.at[sl], si.at[s]).wait()

    def add_roff(s):
        for i in range(NG):
            sl = pl.ds(s * CB + i * 16, 16)
            scb[sl] = scb[sl] + roff

    def gather_issue(c, s, b):
        pltpu.async_copy(xlh.at[scb.at[pl.ds(s * CB, CB)]], rows.at[b], sg.at[b])

    ring_issue(0, 0, sync=True)
    add_roff(0)
    gather_issue(0, 0, 0)
    ring_issue(1, 1)
    ring_issue(2, 2)

    def sixpack(g, _):
        for b6 in range(6):
            c = 6 * g + b6
            b = b6 % 2
            nb = 1 - b
            s = b6 % 3

            @pl.when(c + 1 < NCH_B)
            def _():
                sn = (b6 + 1) % 3
                ring_wait(c + 1, sn)
                add_roff(sn)
                gather_issue(c + 1, sn, nb)

            pltpu.make_async_copy(
                xlh.at[scb.at[pl.ds(s * CB, CB)]], rows.at[b], sg.at[b]).wait()

            @pl.when(c >= 2)
            def _():
                pltpu.make_async_copy(
                    sbuf.at[b], acc_sh.at[dstc.at[b]], ss.at[b]).wait()

            def scale_e(e, _):
                we = plsc.load_gather(wcb, [zero16 + (s * CB + e)])
                for k in range(8):
                    sl = pl.ds(k * 16, 16)
                    sbuf[b, e, sl] = rows[b, e, sl] * we
                return 0

            lax.fori_loop(0, CB, scale_e, 0)
            for i in range(NG):
                dstc[b, pl.ds(i * 16, 16)] = dcb[pl.ds(s * CB + i * 16, 16)]

            pltpu.async_copy(sbuf.at[b], acc_sh.at[dstc.at[b]], ss.at[b], add=True)

            @pl.when(c + 3 < NCH_B)
            def _():
                ring_issue(c + 3, s)

        return 0

    lax.fori_loop(0, NCH_B // 6, sixpack, 0)
    for b in range(2):
        pltpu.make_async_copy(sbuf.at[b], acc_sh.at[dstc.at[b]], ss.at[b]).wait()
    plsc.subcore_barrier()

    pltpu.sync_copy(s2.at[pl.ds(sid * 640, 640)], s0_v)
    pltpu.sync_copy(s2.at[pl.ds(XPAD + sid * 640, 640)], iv_v)
    for i in range(40):
        sl = pl.ds(i * 16, 16)
        iv_v[sl] = 1.0 / (s0_v[sl] + iv_v[sl] + 1e-16)

    def rowblk(rc, _):
        pltpu.sync_copy(acc_sh.at[pl.ds(sid * 640 + rc * 64, 64)], rows.at[0])

        def rowfix(r, _):
            ivr = plsc.load_gather(iv_v, [zero16 + (rc * 64 + r)])
            for k in range(8):
                sl = pl.ds(k * 16, 16)
                rows[0, r, sl] = rows[0, r, sl] * ivr + bias_v[sl]
            return 0

        lax.fori_loop(0, 64, rowfix, 0)
        pltpu.sync_copy(rows.at[0], out.at[pl.ds(roff + sid * 640 + rc * 64, 64)])
        return 0

    lax.fori_loop(0, 10, rowblk, 0)


def _run_agg(xlh, src, dst, w, s2, bias):
    return pl.kernel(
        _agg_body,
        out_type=jax.ShapeDtypeStruct((2 * XPAD, 128), jnp.float32),
        mesh=_MESH,
        scratch_types=[
            pltpu.VMEM((2, CB, 128), jnp.float32),
            pltpu.VMEM((2, CB, 128), jnp.float32),
            pltpu.VMEM((3 * CB,), jnp.int32),
            pltpu.VMEM((3 * CB,), jnp.int32),
            pltpu.VMEM((3 * CB,), jnp.float32),
            pltpu.VMEM((2, CB), jnp.int32),
            pltpu.VMEM((640,), jnp.float32),
            pltpu.VMEM((640,), jnp.float32),
            pltpu.VMEM((128,), jnp.float32),
            pltpu.VMEM_SHARED((XPAD, 128), jnp.float32),
            pltpu.SemaphoreType.DMA((3,)),
            pltpu.SemaphoreType.DMA((2,)),
            pltpu.SemaphoreType.DMA((2,)),
        ],
        compiler_params=_SC_PARAMS,
    )(xlh, src, dst, w, s2, bias)


@jax.jit
def kernel(x, edge_index, W_l, b_l, W_r, b_r, att, bias):
    n = x.shape[0]
    x_pad = jnp.pad(x, ((0, XPAD - n), (0, 0)))
    x_l, x_r = _project(x_pad, W_l, b_l, W_r, b_r)

    loop = jnp.arange(n, dtype=jnp.int32)
    padv = jnp.full((E_PAD - E_TOT,), PAD_IDX, dtype=jnp.int32)
    src = jnp.concatenate([edge_index[0].astype(jnp.int32), loop, padv])
    dst = jnp.concatenate([edge_index[1].astype(jnp.int32), loop, padv])

    w, s2 = _run_logits(x_l, x_r, src, dst, att)

    xlh = jnp.concatenate([x_l[:, :128], x_l[:, 128:]], axis=0)
    o = _run_agg(xlh, src, dst, w, s2, bias)
    return jnp.concatenate([o[:n], o[XPAD:XPAD + n]], axis=1)


# speedup vs baseline: 1.2960x; 1.2960x over previous
import functools

import jax
import jax.numpy as jnp
from jax import lax
from jax.experimental import pallas as pl
from jax.experimental.pallas import tpu as pltpu
from jax.experimental.pallas import tpu_sc as plsc

N_NODES = 10000
D = 256
XPAD = 10240
E_TOT = N_NODES + 160000

CA = 96
NCH_A = 56
EPW_A = CA * NCH_A
E_PAD = 32 * EPW_A

CB = 96
NCH_B = 112
EPW_B = CB * NCH_B

PAD_IDX = 10008

_MESH = plsc.VectorSubcoreMesh(core_axis_name="c", subcore_axis_name="s")
_SC_PARAMS = pltpu.CompilerParams(needs_layout_passes=False)


def _proj_body(x_ref, wl_ref, bl_ref, wr_ref, br_ref, xl_ref, xr_ref, xlh_ref):
    xb = x_ref[...]
    hl = jnp.dot(xb, wl_ref[...], preferred_element_type=jnp.float32) + bl_ref[...]
    xl_ref[...] = hl
    xlh_ref[...] = hl
    xr_ref[...] = jnp.dot(xb, wr_ref[...], preferred_element_type=jnp.float32) + br_ref[...]


def _project(x_pad, W_l, b_l, W_r, b_r):
    blk = 1024
    nb = XPAD // blk
    return pl.pallas_call(
        _proj_body,
        grid=(2, nb),
        in_specs=[
            pl.BlockSpec((blk, D), lambda c, i: (i, 0)),
            pl.BlockSpec((D, 128), lambda c, i: (0, c)),
            pl.BlockSpec((128,), lambda c, i: (c,)),
            pl.BlockSpec((D, 128), lambda c, i: (0, c)),
            pl.BlockSpec((128,), lambda c, i: (c,)),
        ],
        out_specs=[
            pl.BlockSpec((blk, 128), lambda c, i: (i, c)),
            pl.BlockSpec((blk, 128), lambda c, i: (i, c)),
            pl.BlockSpec((blk, 128), lambda c, i: (c * nb + i, 0)),
        ],
        out_shape=[
            jax.ShapeDtypeStruct((XPAD, D), jnp.float32),
            jax.ShapeDtypeStruct((XPAD, D), jnp.float32),
            jax.ShapeDtypeStruct((2 * XPAD, 128), jnp.float32),
        ],
    )(x_pad, W_l, b_l, W_r, b_r)


def _logits_body(xl, xr, src, dst, att, w_out, s2_out,
                 idxs_v, idxd_v, xlr, xrr, w2, dstc, att_v, zbuf, tbuf, s_sh,
                 semgl, semgr, sems, semw):
    cid = lax.axis_index("c")
    sid = lax.axis_index("s")
    wid = cid * 16 + sid
    ebase = wid * EPW_A

    for i in range(40):
        zbuf[pl.ds(i * 16, 16)] = jnp.zeros((16,), jnp.float32)
    pltpu.sync_copy(zbuf, s_sh.at[pl.ds(sid * 640, 640)])
    pltpu.sync_copy(att, att_v)
    pltpu.sync_copy(src.at[pl.ds(ebase, EPW_A)], idxs_v)
    pltpu.sync_copy(dst.at[pl.ds(ebase, EPW_A)], idxd_v)
    plsc.subcore_barrier()

    def gather_pair(c, b):
        pltpu.async_copy(xl.at[idxs_v.at[pl.ds(c * CA, CA)]], xlr.at[b], semgl.at[b])
        pltpu.async_copy(xr.at[idxd_v.at[pl.ds(c * CA, CA)]], xrr.at[b], semgr.at[b])

    for b in range(2):
        gather_pair(b, b)

    lanes16 = lax.iota(jnp.int32, 16) * 16

    def pair(g, _):
        for b in range(2):
            c = 2 * g + b
            pltpu.make_async_copy(
                xl.at[idxs_v.at[pl.ds(c * CA, CA)]], xlr.at[b], semgl.at[b]).wait()
            pltpu.make_async_copy(
                xr.at[idxd_v.at[pl.ds(c * CA, CA)]], xrr.at[b], semgr.at[b]).wait()

            @pl.when(g >= 1)
            def _():
                cp = c - 2
                pltpu.make_async_copy(
                    w2.at[b], s_sh.at[dstc.at[b]], sems.at[b]).wait()
                pltpu.make_async_copy(
                    w2.at[b], w_out.at[pl.ds(ebase + cp * CA, CA)], semw.at[b]).wait()

            xb = xlr.at[b]
            rb = xrr.at[b]

            def edge_group(q, _):
                for j in range(16):
                    e = q * 16 + j
                    acc = jnp.zeros((16,), jnp.float32)
                    for k in range(16):
                        z = xb[e, pl.ds(k * 16, 16)] + rb[e, pl.ds(k * 16, 16)]
                        acc = acc + jnp.maximum(z, 0.2 * z) * att_v[pl.ds(k * 16, 16)]
                    plsc.store_scatter(tbuf, [lanes16 + j], acc)
                vec = tbuf[pl.ds(0, 16)]
                for k in range(1, 16):
                    vec = vec + tbuf[pl.ds(k * 16, 16)]
                w2[b, pl.ds(q * 16, 16)] = jnp.exp(vec)
                return 0

            lax.fori_loop(0, CA // 16, edge_group, 0)

            for i in range(CA // 16):
                sl = pl.ds(i * 16, 16)
                dstc[b, sl] = idxd_v[pl.ds(c * CA + i * 16, 16)]
            pltpu.async_copy(
                w2.at[b], s_sh.at[dstc.at[b]], sems.at[b], add=True)
            pltpu.async_copy(
                w2.at[b], w_out.at[pl.ds(ebase + c * CA, CA)], semw.at[b])

            @pl.when(c + 2 < NCH_A)
            def _():
                gather_pair(c + 2, b)

        return 0

    lax.fori_loop(0, NCH_A // 2, pair, 0)

    for b in range(2):
        c = NCH_A - 2 + b
        pltpu.make_async_copy(
            w2.at[b], s_sh.at[dstc.at[b]], sems.at[b]).wait()
        pltpu.make_async_copy(
            w2.at[b], w_out.at[pl.ds(ebase + c * CA, CA)], semw.at[b]).wait()
    plsc.subcore_barrier()

    @pl.when(sid == 0)
    def _():
        pltpu.sync_copy(s_sh, s2_out.at[pl.ds(cid * XPAD, XPAD)])


def _run_logits(xl, xr, src, dst, att):
    return pl.kernel(
        _logits_body,
        out_type=[
            jax.ShapeDtypeStruct((E_PAD,), jnp.float32),
            jax.ShapeDtypeStruct((2 * XPAD,), jnp.float32),
        ],
        mesh=_MESH,
        scratch_types=[
            pltpu.VMEM((EPW_A,), jnp.int32),
            pltpu.VMEM((EPW_A,), jnp.int32),
            pltpu.VMEM((2, CA, D), jnp.float32),
            pltpu.VMEM((2, CA, D), jnp.float32),
            pltpu.VMEM((2, CA), jnp.float32),
            pltpu.VMEM((2, CA), jnp.int32),
            pltpu.VMEM((D,), jnp.float32),
            pltpu.VMEM((640,), jnp.float32),
            pltpu.VMEM((D,), jnp.float32),
            pltpu.VMEM_SHARED((XPAD,), jnp.float32),
            pltpu.SemaphoreType.DMA((2,)),
            pltpu.SemaphoreType.DMA((2,)),
            pltpu.SemaphoreType.DMA((2,)),
            pltpu.SemaphoreType.DMA((2,)),
        ],
        compiler_params=_SC_PARAMS,
    )(xl, xr, src, dst, att)


def _agg_body(xlh, src, dst, w, s2, bias, out,
              idxd_v, wv, rows, scb, dstc, s0_v, iv_v, bias_v,
              acc_sh, si, sg):
    cid = lax.axis_index("c")
    sid = lax.axis_index("s")
    roff = cid * XPAD
    ebase = sid * EPW_B
    NG = CB // 16
    zero16 = jnp.zeros((16,), jnp.int32)

    def zrow(j, _):
        for k in range(8):
            rows[0, j, pl.ds(k * 16, 16)] = jnp.zeros((16,), jnp.float32)
        return 0

    lax.fori_loop(0, CB, zrow, 0)
    for j in range(8):
        pltpu.sync_copy(rows.at[0, pl.ds(0, 80)],
                        acc_sh.at[pl.ds(sid * 640 + j * 80, 80)])
    pltpu.sync_copy(bias.at[pl.ds(cid * 128, 128)], bias_v)
    pltpu.sync_copy(dst.at[pl.ds(ebase, EPW_B)], idxd_v)
    pltpu.sync_copy(w.at[pl.ds(ebase, EPW_B)], wv)
    plsc.subcore_barrier()

    pltpu.sync_copy(src.at[pl.ds(ebase, CB)], scb.at[0])
    for i in range(NG):
        sl = pl.ds(i * 16, 16)
        scb[0, sl] = scb[0, sl] + roff
    pltpu.async_copy(xlh.at[scb.at[0]], rows.at[0], sg.at[0])
    pltpu.async_copy(src.at[pl.ds(ebase + CB, CB)], scb.at[1], si.at[1])

    def pair(g, _):
        for b in range(2):
            c = 2 * g + b
            nb = 1 - b

            @pl.when(c + 1 < NCH_B)
            def _():
                pltpu.make_async_copy(
                    src.at[pl.ds(ebase + (c + 1) * CB, CB)], scb.at[nb],
                    si.at[nb]).wait()
                for i in range(NG):
                    sl = pl.ds(i * 16, 16)
                    scb[nb, sl] = scb[nb, sl] + roff
                pltpu.async_copy(xlh.at[scb.at[nb]], rows.at[nb], sg.at[nb])

            pltpu.make_async_copy(
                xlh.at[scb.at[b]], rows.at[b], sg.at[b]).wait()

            @pl.when(c + 2 < NCH_B)
            def _():
                pltpu.async_copy(
                    src.at[pl.ds(ebase + (c + 2) * CB, CB)], scb.at[b],
                    si.at[b])

            for i in range(NG):
                sl = pl.ds(i * 16, 16)
                dstc[pl.ds(i * 16, 16)] = idxd_v[pl.ds(c * CB + i * 16, 16)]

            def scale_e(e, _):
                we = plsc.load_gather(wv, [zero16 + (c * CB + e)])
                for k in range(8):
                    sl = pl.ds(k * 16, 16)
                    rows[b, e, sl] = rows[b, e, sl] * we
                return 0

            lax.fori_loop(0, CB, scale_e, 0)
            pltpu.sync_copy(rows.at[b], acc_sh.at[dstc], add=True)

        return 0

    lax.fori_loop(0, NCH_B // 2, pair, 0)
    plsc.subcore_barrier()

    pltpu.sync_copy(s2.at[pl.ds(sid * 640, 640)], s0_v)
    pltpu.sync_copy(s2.at[pl.ds(XPAD + sid * 640, 640)], iv_v)
    for i in range(40):
        sl = pl.ds(i * 16, 16)
        iv_v[sl] = 1.0 / (s0_v[sl] + iv_v[sl] + 1e-16)

    def rowblk(rc, _):
        pltpu.sync_copy(acc_sh.at[pl.ds(sid * 640 + rc * 80, 80)],
                        rows.at[0, pl.ds(0, 80)])

        def rowfix(r, _):
            ivr = plsc.load_gather(iv_v, [zero16 + (rc * 80 + r)])
            for k in range(8):
                sl = pl.ds(k * 16, 16)
                rows[0, r, sl] = rows[0, r, sl] * ivr + bias_v[sl]
            return 0

        lax.fori_loop(0, 80, rowfix, 0)
        pltpu.sync_copy(rows.at[0, pl.ds(0, 80)],
                        out.at[pl.ds(roff + sid * 640 + rc * 80, 80)])
        return 0

    lax.fori_loop(0, 8, rowblk, 0)


def _run_agg(xlh, src, dst, w, s2, bias):
    return pl.kernel(
        _agg_body,
        out_type=jax.ShapeDtypeStruct((2 * XPAD, 128), jnp.float32),
        mesh=_MESH,
        scratch_types=[
            pltpu.VMEM((EPW_B,), jnp.int32),
            pltpu.VMEM((EPW_B,), jnp.float32),
            pltpu.VMEM((2, CB, 128), jnp.float32),
            pltpu.VMEM((2, CB), jnp.int32),
            pltpu.VMEM((CB,), jnp.int32),
            pltpu.VMEM((640,), jnp.float32),
            pltpu.VMEM((640,), jnp.float32),
            pltpu.VMEM((128,), jnp.float32),
            pltpu.VMEM_SHARED((XPAD, 128), jnp.float32),
            pltpu.SemaphoreType.DMA((2,)),
            pltpu.SemaphoreType.DMA((2,)),
        ],
        compiler_params=_SC_PARAMS,
    )(xlh, src, dst, w, s2, bias)


@jax.jit
def kernel(x, edge_index, W_l, b_l, W_r, b_r, att, bias):
    n = x.shape[0]
    x_pad = jnp.pad(x, ((0, XPAD - n), (0, 0)))
    x_l, x_r, xlh = _project(x_pad, W_l, b_l, W_r, b_r)

    loop = jnp.arange(n, dtype=jnp.int32)
    padv = jnp.full((E_PAD - E_TOT,), PAD_IDX, dtype=jnp.int32)
    src = jnp.concatenate([edge_index[0].astype(jnp.int32), loop, padv])
    dst = jnp.concatenate([edge_index[1].astype(jnp.int32), loop, padv])

    w, s2 = _run_logits(x_l, x_r, src, dst, att)
    o = _run_agg(xlh, src, dst, w, s2, bias)
    return jnp.concatenate([o[:n], o[XPAD:XPAD + n]], axis=1)


# speedup vs baseline: 1.6641x; 1.2840x over previous
import functools

import jax
import jax.numpy as jnp
from jax import lax
from jax.experimental import pallas as pl
from jax.experimental.pallas import tpu as pltpu
from jax.experimental.pallas import tpu_sc as plsc

N_NODES = 10000
D = 256
XPAD = 10240
E_TOT = N_NODES + 160000

CA = 96
NCH_A = 56
EPW_A = CA * NCH_A
E_PAD = 32 * EPW_A

CB = 128
NCH_B = 84
EPW_B = CB * NCH_B

PAD_IDX = 10008

_MESH = plsc.VectorSubcoreMesh(core_axis_name="c", subcore_axis_name="s")
_SC_PARAMS = pltpu.CompilerParams(needs_layout_passes=False)


def _proj_body(x_ref, wl_ref, bl_ref, wr_ref, br_ref, xl_ref, xr_ref, xlh_ref):
    xb = x_ref[...]
    hl = jnp.dot(xb, wl_ref[...], preferred_element_type=jnp.float32) + bl_ref[...]
    xl_ref[...] = hl
    xlh_ref[...] = hl
    xr_ref[...] = jnp.dot(xb, wr_ref[...], preferred_element_type=jnp.float32) + br_ref[...]


def _project(x_pad, W_l, b_l, W_r, b_r):
    blk = 1024
    nb = XPAD // blk
    return pl.pallas_call(
        _proj_body,
        grid=(2, nb),
        in_specs=[
            pl.BlockSpec((blk, D), lambda c, i: (i, 0)),
            pl.BlockSpec((D, 128), lambda c, i: (0, c)),
            pl.BlockSpec((128,), lambda c, i: (c,)),
            pl.BlockSpec((D, 128), lambda c, i: (0, c)),
            pl.BlockSpec((128,), lambda c, i: (c,)),
        ],
        out_specs=[
            pl.BlockSpec((blk, 128), lambda c, i: (i, c)),
            pl.BlockSpec((blk, 128), lambda c, i: (i, c)),
            pl.BlockSpec((blk, 128), lambda c, i: (c * nb + i, 0)),
        ],
        out_shape=[
            jax.ShapeDtypeStruct((XPAD, D), jnp.float32),
            jax.ShapeDtypeStruct((XPAD, D), jnp.float32),
            jax.ShapeDtypeStruct((2 * XPAD, 128), jnp.float32),
        ],
    )(x_pad, W_l, b_l, W_r, b_r)


def _logits_body(xl, xr, src, dst, att, w_out, s2_out,
                 idxs_v, idxd_v, xlr, xrr, w2, dstc, att_v, zbuf, tbuf, s_sh,
                 semgl, semgr, sems, semw):
    cid = lax.axis_index("c")
    sid = lax.axis_index("s")
    wid = cid * 16 + sid
    ebase = wid * EPW_A

    for i in range(40):
        zbuf[pl.ds(i * 16, 16)] = jnp.zeros((16,), jnp.float32)
    pltpu.sync_copy(zbuf, s_sh.at[pl.ds(sid * 640, 640)])
    pltpu.sync_copy(att, att_v)
    pltpu.sync_copy(src.at[pl.ds(ebase, EPW_A)], idxs_v)
    pltpu.sync_copy(dst.at[pl.ds(ebase, EPW_A)], idxd_v)
    plsc.subcore_barrier()

    def gather_pair(c, b):
        pltpu.async_copy(xl.at[idxs_v.at[pl.ds(c * CA, CA)]], xlr.at[b], semgl.at[b])
        pltpu.async_copy(xr.at[idxd_v.at[pl.ds(c * CA, CA)]], xrr.at[b], semgr.at[b])

    for b in range(2):
        gather_pair(b, b)

    lanes16 = lax.iota(jnp.int32, 16) * 16

    def pair(g, _):
        for b in range(2):
            c = 2 * g + b
            pltpu.make_async_copy(
                xl.at[idxs_v.at[pl.ds(c * CA, CA)]], xlr.at[b], semgl.at[b]).wait()
            pltpu.make_async_copy(
                xr.at[idxd_v.at[pl.ds(c * CA, CA)]], xrr.at[b], semgr.at[b]).wait()

            @pl.when(g >= 1)
            def _():
                cp = c - 2
                pltpu.make_async_copy(
                    w2.at[b], s_sh.at[dstc.at[b]], sems.at[b]).wait()
                pltpu.make_async_copy(
                    w2.at[b], w_out.at[pl.ds(ebase + cp * CA, CA)], semw.at[b]).wait()

            xb = xlr.at[b]
            rb = xrr.at[b]

            def edge_group(q, _):
                for j in range(16):
                    e = q * 16 + j
                    acc = jnp.zeros((16,), jnp.float32)
                    for k in range(16):
                        z = xb[e, pl.ds(k * 16, 16)] + rb[e, pl.ds(k * 16, 16)]
                        acc = acc + jnp.maximum(z, 0.2 * z) * att_v[pl.ds(k * 16, 16)]
                    plsc.store_scatter(tbuf, [lanes16 + j], acc)
                vec = tbuf[pl.ds(0, 16)]
                for k in range(1, 16):
                    vec = vec + tbuf[pl.ds(k * 16, 16)]
                w2[b, pl.ds(q * 16, 16)] = jnp.exp(vec)
                return 0

            lax.fori_loop(0, CA // 16, edge_group, 0)

            for i in range(CA // 16):
                sl = pl.ds(i * 16, 16)
                dstc[b, sl] = idxd_v[pl.ds(c * CA + i * 16, 16)]
            pltpu.async_copy(
                w2.at[b], s_sh.at[dstc.at[b]], sems.at[b], add=True)
            pltpu.async_copy(
                w2.at[b], w_out.at[pl.ds(ebase + c * CA, CA)], semw.at[b])

            @pl.when(c + 2 < NCH_A)
            def _():
                gather_pair(c + 2, b)

        return 0

    lax.fori_loop(0, NCH_A // 2, pair, 0)

    for b in range(2):
        c = NCH_A - 2 + b
        pltpu.make_async_copy(
            w2.at[b], s_sh.at[dstc.at[b]], sems.at[b]).wait()
        pltpu.make_async_copy(
            w2.at[b], w_out.at[pl.ds(ebase + c * CA, CA)], semw.at[b]).wait()
    plsc.subcore_barrier()

    @pl.when(sid == 0)
    def _():
        pltpu.sync_copy(s_sh, s2_out.at[pl.ds(cid * XPAD, XPAD)])


def _run_logits(xl, xr, src, dst, att):
    return pl.kernel(
        _logits_body,
        out_type=[
            jax.ShapeDtypeStruct((E_PAD,), jnp.float32),
            jax.ShapeDtypeStruct((2 * XPAD,), jnp.float32),
        ],
        mesh=_MESH,
        scratch_types=[
            pltpu.VMEM((EPW_A,), jnp.int32),
            pltpu.VMEM((EPW_A,), jnp.int32),
            pltpu.VMEM((2, CA, D), jnp.float32),
            pltpu.VMEM((2, CA, D), jnp.float32),
            pltpu.VMEM((2, CA), jnp.float32),
            pltpu.VMEM((2, CA), jnp.int32),
            pltpu.VMEM((D,), jnp.float32),
            pltpu.VMEM((640,), jnp.float32),
            pltpu.VMEM((D,), jnp.float32),
            pltpu.VMEM_SHARED((XPAD,), jnp.float32),
            pltpu.SemaphoreType.DMA((2,)),
            pltpu.SemaphoreType.DMA((2,)),
            pltpu.SemaphoreType.DMA((2,)),
            pltpu.SemaphoreType.DMA((2,)),
        ],
        compiler_params=_SC_PARAMS,
    )(xl, xr, src, dst, att)


def _agg_body(xlh, src, dst, w, s2, bias, out,
              wv, rows, scb, dcb, dstc, s0_v, iv_v, bias_v,
              acc_sh, si, sg):
    cid = lax.axis_index("c")
    sid = lax.axis_index("s")
    roff = cid * XPAD
    ebase = sid * EPW_B
    NG = CB // 16
    zero16 = jnp.zeros((16,), jnp.int32)

    def zrow(j, _):
        for k in range(8):
            rows[0, j, pl.ds(k * 16, 16)] = jnp.zeros((16,), jnp.float32)
        return 0

    lax.fori_loop(0, CB, zrow, 0)
    for j in range(8):
        pltpu.sync_copy(rows.at[0, pl.ds(0, 80)],
                        acc_sh.at[pl.ds(sid * 640 + j * 80, 80)])
    pltpu.sync_copy(bias.at[pl.ds(cid * 128, 128)], bias_v)
    pltpu.sync_copy(w.at[pl.ds(ebase, EPW_B)], wv)
    plsc.subcore_barrier()

    pltpu.sync_copy(src.at[pl.ds(ebase, CB)], scb.at[0])
    pltpu.sync_copy(dst.at[pl.ds(ebase, CB)], dcb.at[0])
    for i in range(NG):
        sl = pl.ds(i * 16, 16)
        scb[0, sl] = scb[0, sl] + roff
    pltpu.async_copy(xlh.at[scb.at[0]], rows.at[0], sg.at[0])
    pltpu.async_copy(src.at[pl.ds(ebase + CB, CB)], scb.at[1], si.at[1])
    pltpu.async_copy(dst.at[pl.ds(ebase + CB, CB)], dcb.at[1], si.at[1])

    def pair(g, _):
        for b in range(2):
            c = 2 * g + b
            nb = 1 - b

            @pl.when(c + 1 < NCH_B)
            def _():
                pltpu.make_async_copy(
                    src.at[pl.ds(ebase + (c + 1) * CB, CB)], scb.at[nb],
                    si.at[nb]).wait()
                pltpu.make_async_copy(
                    dst.at[pl.ds(ebase + (c + 1) * CB, CB)], dcb.at[nb],
                    si.at[nb]).wait()
                for i in range(NG):
                    sl = pl.ds(i * 16, 16)
                    scb[nb, sl] = scb[nb, sl] + roff
                pltpu.async_copy(xlh.at[scb.at[nb]], rows.at[nb], sg.at[nb])

            pltpu.make_async_copy(
                xlh.at[scb.at[b]], rows.at[b], sg.at[b]).wait()

            for i in range(NG):
                sl = pl.ds(i * 16, 16)
                dstc[sl] = dcb[b, sl]

            @pl.when(c + 2 < NCH_B)
            def _():
                pltpu.async_copy(
                    src.at[pl.ds(ebase + (c + 2) * CB, CB)], scb.at[b],
                    si.at[b])
                pltpu.async_copy(
                    dst.at[pl.ds(ebase + (c + 2) * CB, CB)], dcb.at[b],
                    si.at[b])

            def scale_e(e, _):
                we = plsc.load_gather(wv, [zero16 + (c * CB + e)])
                for k in range(8):
                    sl = pl.ds(k * 16, 16)
                    rows[b, e, sl] = rows[b, e, sl] * we
                return 0

            lax.fori_loop(0, CB, scale_e, 0)
            pltpu.sync_copy(rows.at[b], acc_sh.at[dstc], add=True)

        return 0

    lax.fori_loop(0, NCH_B // 2, pair, 0)
    plsc.subcore_barrier()

    pltpu.sync_copy(s2.at[pl.ds(sid * 640, 640)], s0_v)
    pltpu.sync_copy(s2.at[pl.ds(XPAD + sid * 640, 640)], iv_v)
    for i in range(40):
        sl = pl.ds(i * 16, 16)
        iv_v[sl] = 1.0 / (s0_v[sl] + iv_v[sl] + 1e-16)

    def rowblk(rc, _):
        pltpu.sync_copy(acc_sh.at[pl.ds(sid * 640 + rc * 80, 80)],
                        rows.at[0, pl.ds(0, 80)])

        def rowfix(r, _):
            ivr = plsc.load_gather(iv_v, [zero16 + (rc * 80 + r)])
            for k in range(8):
                sl = pl.ds(k * 16, 16)
                rows[0, r, sl] = rows[0, r, sl] * ivr + bias_v[sl]
            return 0

        lax.fori_loop(0, 80, rowfix, 0)
        pltpu.sync_copy(rows.at[0, pl.ds(0, 80)],
                        out.at[pl.ds(roff + sid * 640 + rc * 80, 80)])
        return 0

    lax.fori_loop(0, 8, rowblk, 0)


def _run_agg(xlh, src, dst, w, s2, bias):
    return pl.kernel(
        _agg_body,
        out_type=jax.ShapeDtypeStruct((2 * XPAD, 128), jnp.float32),
        mesh=_MESH,
        scratch_types=[
            pltpu.VMEM((EPW_B,), jnp.float32),
            pltpu.VMEM((2, CB, 128), jnp.float32),
            pltpu.VMEM((2, CB), jnp.int32),
            pltpu.VMEM((2, CB), jnp.int32),
            pltpu.VMEM((CB,), jnp.int32),
            pltpu.VMEM((640,), jnp.float32),
            pltpu.VMEM((640,), jnp.float32),
            pltpu.VMEM((128,), jnp.float32),
            pltpu.VMEM_SHARED((XPAD, 128), jnp.float32),
            pltpu.SemaphoreType.DMA((2,)),
            pltpu.SemaphoreType.DMA((2,)),
        ],
        compiler_params=_SC_PARAMS,
    )(xlh, src, dst, w, s2, bias)


@jax.jit
def kernel(x, edge_index, W_l, b_l, W_r, b_r, att, bias):
    n = x.shape[0]
    x_pad = jnp.pad(x, ((0, XPAD - n), (0, 0)))
    x_l, x_r, xlh = _project(x_pad, W_l, b_l, W_r, b_r)

    loop = jnp.arange(n, dtype=jnp.int32)
    npad = E_PAD - E_TOT
    padv = PAD_IDX + (jnp.arange(npad, dtype=jnp.int32) % (XPAD - PAD_IDX))
    src = jnp.concatenate([edge_index[0].astype(jnp.int32), loop, padv])
    dst = jnp.concatenate([edge_index[1].astype(jnp.int32), loop, padv])

    w, s2 = _run_logits(x_l, x_r, src, dst, att)
    o = _run_agg(xlh, src, dst, w, s2, bias)
    return jnp.concatenate([o[:n], o[XPAD:XPAD + n]], axis=1)


# speedup vs baseline: 1.6642x; 1.0001x over previous
import functools

import jax
import jax.numpy as jnp
from jax import lax
from jax.experimental import pallas as pl
from jax.experimental.pallas import tpu as pltpu
from jax.experimental.pallas import tpu_sc as plsc

N_NODES = 10000
D = 256
XPAD = 10240
E_TOT = N_NODES + 160000

CA = 96
NCH_A = 56
EPW_A = CA * NCH_A
E_PAD = 32 * EPW_A

CB = 128
NCH_B = 84
EPW_B = CB * NCH_B

PAD_IDX = 10008

_MESH = plsc.VectorSubcoreMesh(core_axis_name="c", subcore_axis_name="s")
_SC_PARAMS = pltpu.CompilerParams(needs_layout_passes=False)


def _proj_body(x_ref, wl_ref, bl_ref, wr_ref, br_ref, xl_ref, xr_ref, xlh_ref):
    xb = x_ref[...]
    hl = jnp.dot(xb, wl_ref[...], preferred_element_type=jnp.float32) + bl_ref[...]
    xl_ref[...] = hl
    xlh_ref[...] = hl
    xr_ref[...] = jnp.dot(xb, wr_ref[...], preferred_element_type=jnp.float32) + br_ref[...]


def _project(x_pad, W_l, b_l, W_r, b_r):
    blk = 1024
    nb = XPAD // blk
    return pl.pallas_call(
        _proj_body,
        grid=(2, nb),
        in_specs=[
            pl.BlockSpec((blk, D), lambda c, i: (i, 0)),
            pl.BlockSpec((D, 128), lambda c, i: (0, c)),
            pl.BlockSpec((128,), lambda c, i: (c,)),
            pl.BlockSpec((D, 128), lambda c, i: (0, c)),
            pl.BlockSpec((128,), lambda c, i: (c,)),
        ],
        out_specs=[
            pl.BlockSpec((blk, 128), lambda c, i: (i, c)),
            pl.BlockSpec((blk, 128), lambda c, i: (i, c)),
            pl.BlockSpec((blk, 128), lambda c, i: (c * nb + i, 0)),
        ],
        out_shape=[
            jax.ShapeDtypeStruct((XPAD, D), jnp.float32),
            jax.ShapeDtypeStruct((XPAD, D), jnp.float32),
            jax.ShapeDtypeStruct((2 * XPAD, 128), jnp.float32),
        ],
    )(x_pad, W_l, b_l, W_r, b_r)


def _logits_body(xl, xr, src, dst, att, w_out, s2_out,
                 idxs_v, idxd_v, xlr, xrr, w2, dstc, att_v, zbuf, tbuf, s_sh,
                 semgl, semgr, sems, semw):
    cid = lax.axis_index("c")
    sid = lax.axis_index("s")
    wid = cid * 16 + sid
    ebase = wid * EPW_A

    for i in range(40):
        zbuf[pl.ds(i * 16, 16)] = jnp.zeros((16,), jnp.float32)
    pltpu.sync_copy(zbuf, s_sh.at[pl.ds(sid * 640, 640)])
    pltpu.sync_copy(att, att_v)
    pltpu.sync_copy(src.at[pl.ds(ebase, EPW_A)], idxs_v)
    pltpu.sync_copy(dst.at[pl.ds(ebase, EPW_A)], idxd_v)
    plsc.subcore_barrier()

    def gather_pair(c, b):
        pltpu.async_copy(xl.at[idxs_v.at[pl.ds(c * CA, CA)]], xlr.at[b], semgl.at[b])
        pltpu.async_copy(xr.at[idxd_v.at[pl.ds(c * CA, CA)]], xrr.at[b], semgr.at[b])

    for b in range(2):
        gather_pair(b, b)

    lanes16 = lax.iota(jnp.int32, 16) * 16

    def pair(g, _):
        for b in range(2):
            c = 2 * g + b
            pltpu.make_async_copy(
                xl.at[idxs_v.at[pl.ds(c * CA, CA)]], xlr.at[b], semgl.at[b]).wait()
            pltpu.make_async_copy(
                xr.at[idxd_v.at[pl.ds(c * CA, CA)]], xrr.at[b], semgr.at[b]).wait()

            @pl.when(g >= 1)
            def _():
                cp = c - 2
                pltpu.make_async_copy(
                    w2.at[b], s_sh.at[dstc.at[b]], sems.at[b]).wait()
                pltpu.make_async_copy(
                    w2.at[b], w_out.at[pl.ds(ebase + cp * CA, CA)], semw.at[b]).wait()

            xb = xlr.at[b]
            rb = xrr.at[b]

            def edge_group(q, _):
                for j in range(16):
                    e = q * 16 + j
                    acc = jnp.zeros((16,), jnp.float32)
                    for k in range(16):
                        z = xb[e, pl.ds(k * 16, 16)] + rb[e, pl.ds(k * 16, 16)]
                        acc = acc + jnp.maximum(z, 0.2 * z) * att_v[pl.ds(k * 16, 16)]
                    plsc.store_scatter(tbuf, [lanes16 + j], acc)
                vec = tbuf[pl.ds(0, 16)]
                for k in range(1, 16):
                    vec = vec + tbuf[pl.ds(k * 16, 16)]
                w2[b, pl.ds(q * 16, 16)] = jnp.exp(vec)
                return 0

            lax.fori_loop(0, CA // 16, edge_group, 0)

            for i in range(CA // 16):
                sl = pl.ds(i * 16, 16)
                dstc[b, sl] = idxd_v[pl.ds(c * CA + i * 16, 16)]
            pltpu.async_copy(
                w2.at[b], s_sh.at[dstc.at[b]], sems.at[b], add=True)
            pltpu.async_copy(
                w2.at[b], w_out.at[pl.ds(ebase + c * CA, CA)], semw.at[b])

            @pl.when(c + 2 < NCH_A)
            def _():
                gather_pair(c + 2, b)

        return 0

    lax.fori_loop(0, NCH_A // 2, pair, 0)

    for b in range(2):
        c = NCH_A - 2 + b
        pltpu.make_async_copy(
            w2.at[b], s_sh.at[dstc.at[b]], sems.at[b]).wait()
        pltpu.make_async_copy(
            w2.at[b], w_out.at[pl.ds(ebase + c * CA, CA)], semw.at[b]).wait()
    plsc.subcore_barrier()

    @pl.when(sid == 0)
    def _():
        pltpu.sync_copy(s_sh, s2_out.at[pl.ds(cid * XPAD, XPAD)])


def _run_logits(xl, xr, src, dst, att):
    return pl.kernel(
        _logits_body,
        out_type=[
            jax.ShapeDtypeStruct((E_PAD,), jnp.float32),
            jax.ShapeDtypeStruct((2 * XPAD,), jnp.float32),
        ],
        mesh=_MESH,
        scratch_types=[
            pltpu.VMEM((EPW_A,), jnp.int32),
            pltpu.VMEM((EPW_A,), jnp.int32),
            pltpu.VMEM((2, CA, D), jnp.float32),
            pltpu.VMEM((2, CA, D), jnp.float32),
            pltpu.VMEM((2, CA), jnp.float32),
            pltpu.VMEM((2, CA), jnp.int32),
            pltpu.VMEM((D,), jnp.float32),
            pltpu.VMEM((640,), jnp.float32),
            pltpu.VMEM((D,), jnp.float32),
            pltpu.VMEM_SHARED((XPAD,), jnp.float32),
            pltpu.SemaphoreType.DMA((2,)),
            pltpu.SemaphoreType.DMA((2,)),
            pltpu.SemaphoreType.DMA((2,)),
            pltpu.SemaphoreType.DMA((2,)),
        ],
        compiler_params=_SC_PARAMS,
    )(xl, xr, src, dst, att)


def _agg_body(xlh, src, dst, w, s2, bias, out,
              wv, rows, scb, dcb, dstc, s0_v, iv_v, bias_v,
              acc_sh, si, sg):
    cid = lax.axis_index("c")
    sid = lax.axis_index("s")
    roff = cid * XPAD
    ebase = sid * EPW_B
    NG = CB // 16
    zero16 = jnp.zeros((16,), jnp.int32)

    def zrow(j, _):
        for k in range(8):
            rows[0, j, pl.ds(k * 16, 16)] = jnp.zeros((16,), jnp.float32)
        return 0

    lax.fori_loop(0, CB, zrow, 0)
    for j in range(8):
        pltpu.sync_copy(rows.at[0, pl.ds(0, 80)],
                        acc_sh.at[pl.ds(sid * 640 + j * 80, 80)])
    pltpu.sync_copy(bias.at[pl.ds(cid * 128, 128)], bias_v)
    pltpu.sync_copy(w.at[pl.ds(ebase, EPW_B)], wv)
    plsc.subcore_barrier()

    pltpu.sync_copy(src.at[pl.ds(ebase, CB)], scb.at[0])
    pltpu.sync_copy(dst.at[pl.ds(ebase, CB)], dcb.at[0])
    for i in range(NG):
        sl = pl.ds(i * 16, 16)
        scb[0, sl] = scb[0, sl] + roff
    pltpu.async_copy(xlh.at[scb.at[0]], rows.at[0], sg.at[0])
    pltpu.async_copy(src.at[pl.ds(ebase + CB, CB)], scb.at[1], si.at[1])
    pltpu.async_copy(dst.at[pl.ds(ebase + CB, CB)], dcb.at[1], si.at[1])

    def pair(g, _):
        for b in range(2):
            c = 2 * g + b
            nb = 1 - b

            @pl.when(c + 1 < NCH_B)
            def _():
                pltpu.make_async_copy(
                    src.at[pl.ds(ebase + (c + 1) * CB, CB)], scb.at[nb],
                    si.at[nb]).wait()
                pltpu.make_async_copy(
                    dst.at[pl.ds(ebase + (c + 1) * CB, CB)], dcb.at[nb],
                    si.at[nb]).wait()
                for i in range(NG):
                    sl = pl.ds(i * 16, 16)
                    scb[nb, sl] = scb[nb, sl] + roff
                pltpu.async_copy(xlh.at[scb.at[nb]], rows.at[nb], sg.at[nb])

            pltpu.make_async_copy(
                xlh.at[scb.at[b]], rows.at[b], sg.at[b]).wait()

            for i in range(NG):
                sl = pl.ds(i * 16, 16)
                dstc[sl] = dcb[b, sl]

            @pl.when(c + 2 < NCH_B)
            def _():
                pltpu.async_copy(
                    src.at[pl.ds(ebase + (c + 2) * CB, CB)], scb.at[b],
                    si.at[b])
                pltpu.async_copy(
                    dst.at[pl.ds(ebase + (c + 2) * CB, CB)], dcb.at[b],
                    si.at[b])

            def scale_e(e, _):
                we = plsc.load_gather(wv, [zero16 + (c * CB + e)])
                for k in range(8):
                    sl = pl.ds(k * 16, 16)
                    rows[b, e, sl] = rows[b, e, sl] * we
                return 0

            lax.fori_loop(0, CB, scale_e, 0)
            pltpu.sync_copy(rows.at[b], acc_sh.at[dstc], add=True)

        return 0

    lax.fori_loop(0, NCH_B // 2, pair, 0)
    plsc.subcore_barrier()

    pltpu.sync_copy(s2.at[pl.ds(sid * 640, 640)], s0_v)
    pltpu.sync_copy(s2.at[pl.ds(XPAD + sid * 640, 640)], iv_v)
    for i in range(40):
        sl = pl.ds(i * 16, 16)
        iv_v[sl] = 1.0 / (s0_v[sl] + iv_v[sl] + 1e-16)

    def rowblk(rc, _):
        pltpu.sync_copy(acc_sh.at[pl.ds(sid * 640 + rc * 80, 80)],
                        rows.at[0, pl.ds(0, 80)])

        def rowfix(r, _):
            ivr = plsc.load_gather(iv_v, [zero16 + (rc * 80 + r)])
            for k in range(8):
                sl = pl.ds(k * 16, 16)
                rows[0, r, sl] = rows[0, r, sl] * ivr + bias_v[sl]
            return 0

        lax.fori_loop(0, 80, rowfix, 0)
        pltpu.sync_copy(rows.at[0, pl.ds(0, 80)],
                        out.at[pl.ds(sid * 640 + rc * 80, 80),
                               pl.ds(cid * 128, 128)])
        return 0

    lax.fori_loop(0, 8, rowblk, 0)


def _run_agg(xlh, src, dst, w, s2, bias):
    return pl.kernel(
        _agg_body,
        out_type=jax.ShapeDtypeStruct((XPAD, D), jnp.float32),
        mesh=_MESH,
        scratch_types=[
            pltpu.VMEM((EPW_B,), jnp.float32),
            pltpu.VMEM((2, CB, 128), jnp.float32),
            pltpu.VMEM((2, CB), jnp.int32),
            pltpu.VMEM((2, CB), jnp.int32),
            pltpu.VMEM((CB,), jnp.int32),
            pltpu.VMEM((640,), jnp.float32),
            pltpu.VMEM((640,), jnp.float32),
            pltpu.VMEM((128,), jnp.float32),
            pltpu.VMEM_SHARED((XPAD, 128), jnp.float32),
            pltpu.SemaphoreType.DMA((2,)),
            pltpu.SemaphoreType.DMA((2,)),
        ],
        compiler_params=_SC_PARAMS,
    )(xlh, src, dst, w, s2, bias)


@jax.jit
def kernel(x, edge_index, W_l, b_l, W_r, b_r, att, bias):
    n = x.shape[0]
    x_pad = jnp.pad(x, ((0, XPAD - n), (0, 0)))
    x_l, x_r, xlh = _project(x_pad, W_l, b_l, W_r, b_r)

    loop = jnp.arange(n, dtype=jnp.int32)
    npad = E_PAD - E_TOT
    padv = PAD_IDX + (jnp.arange(npad, dtype=jnp.int32) % (XPAD - PAD_IDX))
    src = jnp.concatenate([edge_index[0].astype(jnp.int32), loop, padv])
    dst = jnp.concatenate([edge_index[1].astype(jnp.int32), loop, padv])

    w, s2 = _run_logits(x_l, x_r, src, dst, att)
    o = _run_agg(xlh, src, dst, w, s2, bias)
    return o[:n]
